# baseline (speedup 1.0000x reference)
import functools

import jax
import jax.numpy as jnp
from jax import lax
from jax.experimental import pallas as pl
from jax.experimental.pallas import tpu as pltpu
from jax.experimental.pallas import tpu_sc as plsc

_NUM_CLASSES = 100000
_D = 64
_B = 16384
_LAMBDA_C = 0.003

_INFO = plsc.get_sparse_core_info()
_NC, _NS, _L = _INFO.num_cores, _INFO.num_subcores, _INFO.num_lanes
_NW = _NC * _NS
_ROUNDS = _D // _NW
_FCH = 4096
_NCH = _B // _FCH
_TOT = _ROUNDS * _NCH
_UNROLL = 4


@functools.partial(
    pl.kernel,
    out_type=jax.ShapeDtypeStruct((_NW * _L,), jnp.float32),
    mesh=plsc.VectorSubcoreMesh(core_axis_name="c", subcore_axis_name="s"),
    scratch_types=[
        pltpu.VMEM((_B,), jnp.int32),
        pltpu.VMEM((_NUM_CLASSES,), jnp.float32),
        pltpu.VMEM((2, _FCH), jnp.float32),
        pltpu.VMEM((_L,), jnp.float32),
        pltpu.SemaphoreType.DMA,
        pltpu.SemaphoreType.DMA,
        pltpu.SemaphoreType.DMA,
        pltpu.SemaphoreType.DMA,
    ],
    compiler_params=pltpu.CompilerParams(needs_layout_passes=False),
)
def _center_loss_sc(ft_hbm, idx_hbm, ct_hbm, out_hbm,
                    idx_v, ct_v, ft_v, part_v, isem, csem, fsem0, fsem1):
    wid = lax.axis_index("s") * _NC + lax.axis_index("c")
    fsems = (fsem0, fsem1)

    def dim_of(k):
        return jnp.where(k >= _NCH, wid + _NW, wid)

    def fire_ft(k, buf):
        pltpu.async_copy(
            ft_hbm.at[dim_of(k), pl.ds(lax.rem(k, _NCH) * _FCH, _FCH)],
            ft_v.at[buf], fsems[buf])

    def wait_ct():
        pltpu.make_async_copy(ct_hbm.at[0], ct_v, csem).wait()

    idx_cp = pltpu.async_copy(idx_hbm, idx_v, isem)
    pltpu.async_copy(ct_hbm.at[wid], ct_v, csem)
    idx_cp.wait()
    fire_ft(jnp.int32(0), 0)

    def pair_body(j, accs):
        for b in range(2):
            k = j * 2 + b
            @pl.when(k == 0)
            def _():
                wait_ct()

            @pl.when(k == _NCH)
            def _():
                wait_ct()

            pltpu.make_async_copy(
                ft_hbm.at[0, pl.ds(0, _FCH)], ft_v.at[b], fsems[b]).wait()

            @pl.when(k + 1 < _TOT)
            def _():
                fire_ft(k + 1, (b + 1) % 2)

            def grp_body(i, accs, b=b):
                out = []
                for u in range(_UNROLL):
                    off = i * (_L * _UNROLL) + u * _L
                    labs = idx_v[pl.ds(lax.rem(k, _NCH) * _FCH + off, _L)]
                    c = plsc.load_gather(ct_v, [labs])
                    f = ft_v[b, pl.ds(off, _L)]
                    e = f - c
                    out.append(accs[u] + e * e)
                return tuple(out)

            accs = lax.fori_loop(0, _FCH // (_L * _UNROLL), grp_body, accs)

            @pl.when(k == _NCH - 1)
            def _():
                pltpu.async_copy(ct_hbm.at[wid + _NW], ct_v, csem)

        return accs

    zero = jnp.zeros((_L,), jnp.float32)
    accs = lax.fori_loop(0, _TOT // 2, pair_body, (zero,) * _UNROLL)

    total = (accs[0] + accs[1]) + (accs[2] + accs[3])
    part_v[...] = total * (_LAMBDA_C / float(_B * _D))
    pltpu.sync_copy(part_v, out_hbm.at[pl.ds(wid * _L, _L)])


def kernel(features, labels, centers):
    partials = _center_loss_sc(
        features.T, labels.astype(jnp.int32), centers.T)
    return jnp.sum(partials)


# speedup vs baseline: 1.0022x; 1.0022x over previous
import functools

import jax
import jax.numpy as jnp
from jax import lax
from jax.experimental import pallas as pl
from jax.experimental.pallas import tpu as pltpu
from jax.experimental.pallas import tpu_sc as plsc

_NUM_CLASSES = 100000
_D = 64
_B = 16384
_LAMBDA_C = 0.003

_INFO = plsc.get_sparse_core_info()
_NC, _NS, _L = _INFO.num_cores, _INFO.num_subcores, _INFO.num_lanes
_NW = _NC * _NS
_ROUNDS = _D // _NW
_FCH = 4096
_NCH = _B // _FCH
_TOT = _ROUNDS * _NCH
_UNROLL = 8


@functools.partial(
    pl.kernel,
    out_type=jax.ShapeDtypeStruct((_NW * _L,), jnp.float32),
    mesh=plsc.VectorSubcoreMesh(core_axis_name="c", subcore_axis_name="s"),
    scratch_types=[
        pltpu.VMEM((_B,), jnp.int32),
        pltpu.VMEM((_NUM_CLASSES,), jnp.float32),
        pltpu.VMEM((2, _FCH), jnp.float32),
        pltpu.VMEM((_L,), jnp.float32),
        pltpu.SemaphoreType.DMA,
        pltpu.SemaphoreType.DMA,
        pltpu.SemaphoreType.DMA,
        pltpu.SemaphoreType.DMA,
    ],
    compiler_params=pltpu.CompilerParams(needs_layout_passes=False),
)
def _center_loss_sc(ft_hbm, idx_hbm, ct_hbm, out_hbm,
                    idx_v, ct_v, ft_v, part_v, isem, csem, fsem0, fsem1):
    wid = lax.axis_index("s") * _NC + lax.axis_index("c")
    fsems = (fsem0, fsem1)

    def dim_of(k):
        return jnp.where(k >= _NCH, wid + _NW, wid)

    def fire_ft(k, buf):
        pltpu.async_copy(
            ft_hbm.at[dim_of(k), pl.ds(lax.rem(k, _NCH) * _FCH, _FCH)],
            ft_v.at[buf], fsems[buf])

    def wait_ct():
        pltpu.make_async_copy(ct_hbm.at[0], ct_v, csem).wait()

    idx_cp = pltpu.async_copy(idx_hbm, idx_v, isem)
    pltpu.async_copy(ct_hbm.at[wid], ct_v, csem)
    idx_cp.wait()
    fire_ft(jnp.int32(0), 0)

    def pair_body(j, accs):
        for b in range(2):
            k = j * 2 + b
            @pl.when(k == 0)
            def _():
                wait_ct()

            @pl.when(k == _NCH)
            def _():
                wait_ct()

            pltpu.make_async_copy(
                ft_hbm.at[0, pl.ds(0, _FCH)], ft_v.at[b], fsems[b]).wait()

            @pl.when(k + 1 < _TOT)
            def _():
                fire_ft(k + 1, (b + 1) % 2)

            kbase = lax.rem(k, _NCH) * _FCH

            def grp_body(i, accs, b=b, kbase=kbase):
                out = []
                ibase = i * (_L * _UNROLL)
                for u in range(_UNROLL):
                    off = ibase + u * _L
                    labs = idx_v[pl.ds(kbase + off, _L)]
                    c = plsc.load_gather(ct_v, [labs])
                    f = ft_v[b, pl.ds(off, _L)]
                    e = f - c
                    out.append(accs[u] + e * e)
                return tuple(out)

            accs = lax.fori_loop(0, _FCH // (_L * _UNROLL), grp_body, accs)

            @pl.when(k == _NCH - 1)
            def _():
                pltpu.async_copy(ct_hbm.at[wid + _NW], ct_v, csem)

        return accs

    zero = jnp.zeros((_L,), jnp.float32)
    accs = lax.fori_loop(0, _TOT // 2, pair_body, (zero,) * _UNROLL)

    total = ((accs[0] + accs[1]) + (accs[2] + accs[3])) + (
        (accs[4] + accs[5]) + (accs[6] + accs[7]))
    part_v[...] = total * (_LAMBDA_C / float(_B * _D))
    pltpu.sync_copy(part_v, out_hbm.at[pl.ds(wid * _L, _L)])


def kernel(features, labels, centers):
    partials = _center_loss_sc(
        features.T, labels.astype(jnp.int32), centers.T)
    return jnp.sum(partials)


# speedup vs baseline: 1.0031x; 1.0009x over previous
import functools

import jax
import jax.numpy as jnp
from jax import lax
from jax.experimental import pallas as pl
from jax.experimental.pallas import tpu as pltpu
from jax.experimental.pallas import tpu_sc as plsc

_NUM_CLASSES = 100000
_D = 64
_B = 16384
_LAMBDA_C = 0.003

_INFO = plsc.get_sparse_core_info()
_NC, _NS, _L = _INFO.num_cores, _INFO.num_subcores, _INFO.num_lanes
_NW = _NC * _NS
_ROUNDS = _D // _NW
_FCH = 4096
_NCH = _B // _FCH
_TOT = _ROUNDS * _NCH
_UNROLL = 8


@functools.partial(
    pl.kernel,
    out_type=jax.ShapeDtypeStruct((_NW * _L,), jnp.float32),
    mesh=plsc.VectorSubcoreMesh(core_axis_name="c", subcore_axis_name="s"),
    scratch_types=[
        pltpu.VMEM((_B,), jnp.int32),
        pltpu.VMEM((_NUM_CLASSES,), jnp.float32),
        pltpu.VMEM((2, _FCH), jnp.float32),
        pltpu.VMEM((_L,), jnp.float32),
        pltpu.SemaphoreType.DMA,
        pltpu.SemaphoreType.DMA,
        pltpu.SemaphoreType.DMA,
        pltpu.SemaphoreType.DMA,
    ],
    compiler_params=pltpu.CompilerParams(needs_layout_passes=False),
)
def _center_loss_sc(ft_hbm, idx_hbm, ct_hbm, out_hbm,
                    idx_v, ct_v, ft_v, part_v, isem, csem, fsem0, fsem1):
    wid = lax.axis_index("s") * _NC + lax.axis_index("c")
    fsems = (fsem0, fsem1)

    def dim_of(k):
        return jnp.where(k >= _NCH, wid + _NW, wid)

    def fire_ft(k, buf):
        pltpu.async_copy(
            ft_hbm.at[dim_of(k), pl.ds(lax.rem(k, _NCH) * _FCH, _FCH)],
            ft_v.at[buf], fsems[buf])

    def fire_ct(d):
        pltpu.async_copy(ct_hbm.at[d], ct_v, csem)

    def wait_ct():
        pltpu.make_async_copy(ct_hbm.at[0], ct_v, csem).wait()

    idx_cp = pltpu.async_copy(idx_hbm, idx_v, isem)
    fire_ct(wid)
    idx_cp.wait()
    fire_ft(jnp.int32(0), 0)

    def pair_body(j, accs):
        for b in range(2):
            k = j * 2 + b
            @pl.when(k == 0)
            def _():
                wait_ct()

            @pl.when(k == _NCH)
            def _():
                wait_ct()

            pltpu.make_async_copy(
                ft_hbm.at[0, pl.ds(0, _FCH)], ft_v.at[b], fsems[b]).wait()

            @pl.when(k + 1 < _TOT)
            def _():
                fire_ft(k + 1, (b + 1) % 2)

            kbase = lax.rem(k, _NCH) * _FCH

            def grp_body(i, accs, b=b, kbase=kbase):
                out = []
                ibase = i * (_L * _UNROLL)
                for u in range(_UNROLL):
                    off = ibase + u * _L
                    labs = idx_v[pl.ds(kbase + off, _L)]
                    c = plsc.load_gather(ct_v, [labs])
                    f = ft_v[b, pl.ds(off, _L)]
                    e = f - c
                    out.append(accs[u] + e * e)
                return tuple(out)

            accs = lax.fori_loop(0, _FCH // (_L * _UNROLL), grp_body, accs)

            @pl.when(k == _NCH - 1)
            def _():
                fire_ct(wid + _NW)

        return accs

    zero = jnp.zeros((_L,), jnp.float32)
    accs = lax.fori_loop(0, _TOT // 2, pair_body, (zero,) * _UNROLL)

    total = ((accs[0] + accs[1]) + (accs[2] + accs[3])) + (
        (accs[4] + accs[5]) + (accs[6] + accs[7]))
    part_v[...] = total * (_LAMBDA_C / float(_B * _D))
    pltpu.sync_copy(part_v, out_hbm.at[pl.ds(wid * _L, _L)])


def kernel(features, labels, centers):
    partials = _center_loss_sc(
        features.T, labels.astype(jnp.int32), centers.T)
    return jnp.sum(partials)
